# Initial kernel scaffold; baseline (speedup 1.0000x reference)
import jax
import jax.numpy as jnp
from jax.experimental import pallas as pl


def kernel(x, edge_index, W1, b1, W2, b2, Wl, bl):
    raise NotImplementedError("write your pallas kernel here")


# speedup vs baseline: 12.7808x; 12.7808x over previous
import functools

import jax
import jax.numpy as jnp
from jax import lax
from jax.experimental import pallas as pl
from jax.experimental.pallas import tpu as pltpu
from jax.experimental.pallas import tpu_sc as plsc

N = 10000
E = 320000
D = 16
NC = 2
NS = 16
TILES = NC * NS
CW = 125
CH = E // (TILES * CW)
NP = 10240
RPT = NP // NS
RSTG = 80

_f32 = jnp.float32
_MESH = plsc.VectorSubcoreMesh(
    core_axis_name="c", subcore_axis_name="s", num_cores=NC, num_subcores=NS)


def _zero_acc(stage, acc, s):
  for k in range(RPT // RSTG):
    pltpu.sync_copy(stage, acc.at[pl.ds(s * RPT + k * RSTG, RSTG)])


def _read_acc(stage, acc, out, s):
  for k in range(RPT // RSTG):
    rows = pl.ds(s * RPT + k * RSTG, RSTG)
    pltpu.sync_copy(acc.at[rows], stage)
    pltpu.sync_copy(stage, out.at[rows])


def _fill_rows(ref, nrows, value):
  def body(i, carry):
    ref[i, :] = jnp.full((D,), value, _f32)
    return carry
  lax.fori_loop(0, nrows, body, 0)


@functools.partial(
    pl.kernel,
    mesh=_MESH,
    compiler_params=pltpu.CompilerParams(use_tc_tiling_on_sc=False),
    out_type=[jax.ShapeDtypeStruct((NP, D), _f32)] * 4,
    scratch_types=[
        pltpu.VMEM((CH, CW), jnp.int32),
        pltpu.VMEM((CH, CW), jnp.int32),
        pltpu.VMEM((CW, D), _f32),
        pltpu.VMEM((RSTG, D), _f32),
        pltpu.VMEM_SHARED((NP, D), _f32),
        pltpu.VMEM_SHARED((NP, D), _f32),
    ],
)
def _sc_degrees(src_hbm, dst_hbm, ds0, ds1, dd0, dd1,
                idx_s, idx_d, ones_v, stage, acc_s, acc_d):
  c = lax.axis_index("c")
  s = lax.axis_index("s")
  g = c * NS + s

  pltpu.sync_copy(src_hbm.at[pl.ds(g * CH, CH)], idx_s)
  pltpu.sync_copy(dst_hbm.at[pl.ds(g * CH, CH)], idx_d)
  _fill_rows(ones_v, CW, 1.0)
  _fill_rows(stage, RSTG, 0.0)
  _zero_acc(stage, acc_s, s)
  _zero_acc(stage, acc_d, s)
  plsc.subcore_barrier()

  def body(j, carry):
    pltpu.sync_copy(ones_v, acc_s.at[idx_s.at[j]], add=True)
    pltpu.sync_copy(ones_v, acc_d.at[idx_d.at[j]], add=True)
    return carry
  lax.fori_loop(0, CH, body, 0)
  plsc.subcore_barrier()

  @pl.when(c == 0)
  def _():
    _read_acc(stage, acc_s, ds0, s)
    _read_acc(stage, acc_d, dd0, s)

  @pl.when(c == 1)
  def _():
    _read_acc(stage, acc_s, ds1, s)
    _read_acc(stage, acc_d, dd1, s)


@functools.partial(
    pl.kernel,
    mesh=_MESH,
    compiler_params=pltpu.CompilerParams(use_tc_tiling_on_sc=False),
    out_type=[jax.ShapeDtypeStruct((NP, D), _f32)] * 2,
    scratch_types=[
        pltpu.VMEM((CH, CW), jnp.int32),
        pltpu.VMEM((CH, CW), jnp.int32),
        pltpu.VMEM((CW, D), _f32),
        pltpu.VMEM((RSTG, D), _f32),
        pltpu.VMEM_SHARED((NP, D), _f32),
        pltpu.SemaphoreType.DMA,
    ],
)
def _sc_edge_pass(h_hbm, src_hbm, dst_hbm, p0, p1,
                  idx_s, idx_d, rows_v, stage, acc, sem):
  c = lax.axis_index("c")
  s = lax.axis_index("s")
  g = c * NS + s

  pltpu.sync_copy(src_hbm.at[pl.ds(g * CH, CH)], idx_s)
  pltpu.sync_copy(dst_hbm.at[pl.ds(g * CH, CH)], idx_d)
  _fill_rows(stage, RSTG, 0.0)
  _zero_acc(stage, acc, s)
  plsc.subcore_barrier()

  def body(j, carry):
    pltpu.async_copy(h_hbm.at[idx_s.at[j]], rows_v, sem).wait()
    pltpu.sync_copy(rows_v, acc.at[idx_d.at[j]], add=True)
    return carry
  lax.fori_loop(0, CH, body, 0)
  plsc.subcore_barrier()

  @pl.when(c == 0)
  def _():
    _read_acc(stage, acc, p0, s)

  @pl.when(c == 1)
  def _():
    _read_acc(stage, acc, p1, s)


_BLK = 1000
_GRID = N // _BLK


def _row_spec():
  return pl.BlockSpec((_BLK, D), lambda i: (i, 0))


def _prep_body(ds0, ds1, dd0, dd1, x, w1, h0_o, ns_o, nd_o):
  ns = lax.rsqrt(jnp.maximum(ds0[...] + ds1[...], 1.0))
  nd = lax.rsqrt(jnp.maximum(dd0[...] + dd1[...], 1.0))
  h0_o[...] = jnp.dot(x[...], w1[...], preferred_element_type=_f32) * ns
  ns_o[...] = ns
  nd_o[...] = nd


def _tc_prep(ds0, ds1, dd0, dd1, x, w1):
  return pl.pallas_call(
      _prep_body,
      grid=(_GRID,),
      in_specs=[
          _row_spec(), _row_spec(), _row_spec(), _row_spec(),
          pl.BlockSpec((_BLK, 128), lambda i: (i, 0)),
          pl.BlockSpec((128, D), lambda i: (0, 0)),
      ],
      out_specs=[_row_spec(), _row_spec(), _row_spec()],
      out_shape=[jax.ShapeDtypeStruct((N, D), _f32)] * 3,
  )(ds0, ds1, dd0, dd1, x, w1)


def _mid_body(p0, p1, nd, ns, b1, h1s_o):
  h1 = (p0[...] + p1[...]) * nd[...] + b1[...]
  h1s_o[...] = jnp.maximum(h1, 0.0) * ns[...]


def _tc_mid(p0, p1, nd, ns, b1):
  return pl.pallas_call(
      _mid_body,
      grid=(_GRID,),
      in_specs=[
          _row_spec(), _row_spec(), _row_spec(), _row_spec(),
          pl.BlockSpec((D,), lambda i: (0,)),
      ],
      out_specs=_row_spec(),
      out_shape=jax.ShapeDtypeStruct((N, D), _f32),
  )(p0, p1, nd, ns, b1)


def _final_body(p0, p1, nd, w2, b2, wl, bl, out_o, mx):
  agg = (p0[...] + p1[...]) * nd[...]
  h2 = jnp.dot(agg, w2[...], preferred_element_type=_f32) + b2[...]
  m = jnp.max(h2, axis=0, keepdims=True)
  i = pl.program_id(0)

  @pl.when(i == 0)
  def _():
    mx[...] = m

  @pl.when(i > 0)
  def _():
    mx[...] = jnp.maximum(mx[...], m)

  @pl.when(i == _GRID - 1)
  def _():
    out_o[...] = jnp.dot(mx[...], wl[...], preferred_element_type=_f32) + bl[...]


def _tc_final(p0, p1, nd, w2, b2, wl, bl):
  n_classes = wl.shape[1]
  return pl.pallas_call(
      _final_body,
      grid=(_GRID,),
      in_specs=[
          _row_spec(), _row_spec(), _row_spec(),
          pl.BlockSpec((D, D), lambda i: (0, 0)),
          pl.BlockSpec((D,), lambda i: (0,)),
          pl.BlockSpec((D, n_classes), lambda i: (0, 0)),
          pl.BlockSpec((n_classes,), lambda i: (0,)),
      ],
      out_specs=pl.BlockSpec((1, n_classes), lambda i: (0, 0)),
      out_shape=jax.ShapeDtypeStruct((1, n_classes), _f32),
      scratch_shapes=[pltpu.VMEM((1, D), _f32)],
  )(p0, p1, nd, w2, b2, wl, bl)


def kernel(x, edge_index, W1, b1, W2, b2, Wl, bl):
  src2 = edge_index[0].reshape(E // CW, CW)
  dst2 = edge_index[1].reshape(E // CW, CW)

  ds0, ds1, dd0, dd1 = _sc_degrees(src2, dst2)
  h0, ns, nd = _tc_prep(ds0, ds1, dd0, dd1, x, W1)
  p10, p11 = _sc_edge_pass(h0, src2, dst2)
  h1s = _tc_mid(p10, p11, nd, ns, b1)
  p20, p21 = _sc_edge_pass(h1s, src2, dst2)
  return _tc_final(p20, p21, nd, W2, b2, Wl, bl)


# speedup vs baseline: 19.9436x; 1.5604x over previous
import functools

import jax
import jax.numpy as jnp
from jax import lax
from jax.experimental import pallas as pl
from jax.experimental.pallas import tpu as pltpu
from jax.experimental.pallas import tpu_sc as plsc

N = 10000
E = 320000
D = 16
NC = 2
NS = 16
TILES = NC * NS
CW = 125
CH = E // (TILES * CW)
NP = 10240
RPT = NP // NS
RSTG = 80

_f32 = jnp.float32
_MESH = plsc.VectorSubcoreMesh(
    core_axis_name="c", subcore_axis_name="s", num_cores=NC, num_subcores=NS)


def _zero_acc(stage, acc, s):
  for k in range(RPT // RSTG):
    pltpu.sync_copy(stage, acc.at[pl.ds(s * RPT + k * RSTG, RSTG)])


def _read_acc(stage, acc, out, s):
  for k in range(RPT // RSTG):
    rows = pl.ds(s * RPT + k * RSTG, RSTG)
    pltpu.sync_copy(acc.at[rows], stage)
    pltpu.sync_copy(stage, out.at[rows])


def _fill_rows(ref, nrows, value):
  def body(i, carry):
    ref[i, :] = jnp.full((D,), value, _f32)
    return carry
  lax.fori_loop(0, nrows, body, 0)


@functools.partial(
    pl.kernel,
    mesh=_MESH,
    compiler_params=pltpu.CompilerParams(use_tc_tiling_on_sc=False),
    out_type=[jax.ShapeDtypeStruct((NP, D), _f32)] * 4,
    scratch_types=[
        pltpu.VMEM((CH, CW), jnp.int32),
        pltpu.VMEM((CH, CW), jnp.int32),
        pltpu.VMEM((CW, D), _f32),
        pltpu.VMEM((RSTG, D), _f32),
        pltpu.VMEM_SHARED((NP, D), _f32),
        pltpu.VMEM_SHARED((NP, D), _f32),
        pltpu.SemaphoreType.DMA,
        pltpu.SemaphoreType.DMA,
        pltpu.SemaphoreType.DMA,
        pltpu.SemaphoreType.DMA,
    ],
)
def _sc_degrees(src_hbm, dst_hbm, ds0, ds1, dd0, dd1,
                idx_s, idx_d, ones_v, stage, acc_s, acc_d, sa, sb, sc_, sd):
  c = lax.axis_index("c")
  s = lax.axis_index("s")
  g = c * NS + s

  pltpu.sync_copy(src_hbm.at[pl.ds(g * CH, CH)], idx_s)
  pltpu.sync_copy(dst_hbm.at[pl.ds(g * CH, CH)], idx_d)
  _fill_rows(ones_v, CW, 1.0)
  _fill_rows(stage, RSTG, 0.0)
  _zero_acc(stage, acc_s, s)
  _zero_acc(stage, acc_d, s)
  plsc.subcore_barrier()

  def body(j2, carry):
    j = 2 * j2
    for b, (ss, sdst) in enumerate(((sa, sb), (sc_, sd))):
      @pl.when(j2 > 0)
      def _():
        pltpu.make_async_copy(ones_v, acc_s.at[idx_s.at[0]], ss).wait()
        pltpu.make_async_copy(ones_v, acc_d.at[idx_d.at[0]], sdst).wait()
      pltpu.async_copy(ones_v, acc_s.at[idx_s.at[j + b]], ss, add=True)
      pltpu.async_copy(ones_v, acc_d.at[idx_d.at[j + b]], sdst, add=True)
    return carry
  lax.fori_loop(0, CH // 2, body, 0)
  for ss in (sa, sb, sc_, sd):
    pltpu.make_async_copy(ones_v, acc_s.at[idx_s.at[0]], ss).wait()
  plsc.subcore_barrier()

  @pl.when(c == 0)
  def _():
    _read_acc(stage, acc_s, ds0, s)
    _read_acc(stage, acc_d, dd0, s)

  @pl.when(c == 1)
  def _():
    _read_acc(stage, acc_s, ds1, s)
    _read_acc(stage, acc_d, dd1, s)


@functools.partial(
    pl.kernel,
    mesh=_MESH,
    compiler_params=pltpu.CompilerParams(use_tc_tiling_on_sc=False),
    out_type=[jax.ShapeDtypeStruct((NP, D), _f32)] * 2,
    scratch_types=[
        pltpu.VMEM((CH, CW), jnp.int32),
        pltpu.VMEM((CH, CW), jnp.int32),
        pltpu.VMEM((CW, D), _f32),
        pltpu.VMEM((CW, D), _f32),
        pltpu.VMEM((CW, D), _f32),
        pltpu.VMEM((CW, D), _f32),
        pltpu.VMEM((RSTG, D), _f32),
        pltpu.VMEM_SHARED((NP, D), _f32),
        pltpu.SemaphoreType.DMA,
        pltpu.SemaphoreType.DMA,
        pltpu.SemaphoreType.DMA,
        pltpu.SemaphoreType.DMA,
        pltpu.SemaphoreType.DMA,
        pltpu.SemaphoreType.DMA,
        pltpu.SemaphoreType.DMA,
        pltpu.SemaphoreType.DMA,
    ],
)
def _sc_edge_pass(h_hbm, src_hbm, dst_hbm, p0, p1,
                  idx_s, idx_d, r0, r1, r2, r3, stage, acc,
                  g0, g1, g2, g3, s0, s1, s2, s3):
  c = lax.axis_index("c")
  s = lax.axis_index("s")
  g = c * NS + s

  pltpu.sync_copy(src_hbm.at[pl.ds(g * CH, CH)], idx_s)
  pltpu.sync_copy(dst_hbm.at[pl.ds(g * CH, CH)], idx_d)
  _fill_rows(stage, RSTG, 0.0)
  _zero_acc(stage, acc, s)
  plsc.subcore_barrier()

  rows_bufs = (r0, r1, r2, r3)
  gsems = (g0, g1, g2, g3)
  ssems = (s0, s1, s2, s3)

  for b in range(4):
    pltpu.async_copy(h_hbm.at[idx_s.at[b]], rows_bufs[b], gsems[b])

  def body(j4, carry):
    for b in range(4):
      j = j4 * 4 + b
      pltpu.make_async_copy(h_hbm.at[idx_s.at[0]], rows_bufs[b], gsems[b]).wait()
      pltpu.async_copy(rows_bufs[b], acc.at[idx_d.at[j]], ssems[b], add=True)

      @pl.when(j + 4 < CH)
      def _():
        pltpu.make_async_copy(rows_bufs[b], acc.at[idx_d.at[0]], ssems[b]).wait()
        pltpu.async_copy(h_hbm.at[idx_s.at[j + 4]], rows_bufs[b], gsems[b])
    return carry
  lax.fori_loop(0, CH // 4, body, 0)
  for b in range(4):
    pltpu.make_async_copy(rows_bufs[b], acc.at[idx_d.at[0]], ssems[b]).wait()
  plsc.subcore_barrier()

  @pl.when(c == 0)
  def _():
    _read_acc(stage, acc, p0, s)

  @pl.when(c == 1)
  def _():
    _read_acc(stage, acc, p1, s)


_BLK = 1000
_GRID = N // _BLK


def _row_spec():
  return pl.BlockSpec((_BLK, D), lambda i: (i, 0))


def _prep_body(ds0, ds1, dd0, dd1, x, w1, h0_o, ns_o, nd_o):
  ns = lax.rsqrt(jnp.maximum(ds0[...] + ds1[...], 1.0))
  nd = lax.rsqrt(jnp.maximum(dd0[...] + dd1[...], 1.0))
  h0_o[...] = jnp.dot(x[...], w1[...], preferred_element_type=_f32) * ns
  ns_o[...] = ns
  nd_o[...] = nd


def _tc_prep(ds0, ds1, dd0, dd1, x, w1):
  return pl.pallas_call(
      _prep_body,
      grid=(_GRID,),
      in_specs=[
          _row_spec(), _row_spec(), _row_spec(), _row_spec(),
          pl.BlockSpec((_BLK, 128), lambda i: (i, 0)),
          pl.BlockSpec((128, D), lambda i: (0, 0)),
      ],
      out_specs=[_row_spec(), _row_spec(), _row_spec()],
      out_shape=[jax.ShapeDtypeStruct((N, D), _f32)] * 3,
  )(ds0, ds1, dd0, dd1, x, w1)


def _mid_body(p0, p1, nd, ns, b1, h1s_o):
  h1 = (p0[...] + p1[...]) * nd[...] + b1[...]
  h1s_o[...] = jnp.maximum(h1, 0.0) * ns[...]


def _tc_mid(p0, p1, nd, ns, b1):
  return pl.pallas_call(
      _mid_body,
      grid=(_GRID,),
      in_specs=[
          _row_spec(), _row_spec(), _row_spec(), _row_spec(),
          pl.BlockSpec((D,), lambda i: (0,)),
      ],
      out_specs=_row_spec(),
      out_shape=jax.ShapeDtypeStruct((N, D), _f32),
  )(p0, p1, nd, ns, b1)


def _final_body(p0, p1, nd, w2, b2, wl, bl, out_o, mx):
  agg = (p0[...] + p1[...]) * nd[...]
  h2 = jnp.dot(agg, w2[...], preferred_element_type=_f32) + b2[...]
  m = jnp.max(h2, axis=0, keepdims=True)
  i = pl.program_id(0)

  @pl.when(i == 0)
  def _():
    mx[...] = m

  @pl.when(i > 0)
  def _():
    mx[...] = jnp.maximum(mx[...], m)

  @pl.when(i == _GRID - 1)
  def _():
    out_o[...] = jnp.dot(mx[...], wl[...], preferred_element_type=_f32) + bl[...]


def _tc_final(p0, p1, nd, w2, b2, wl, bl):
  n_classes = wl.shape[1]
  return pl.pallas_call(
      _final_body,
      grid=(_GRID,),
      in_specs=[
          _row_spec(), _row_spec(), _row_spec(),
          pl.BlockSpec((D, D), lambda i: (0, 0)),
          pl.BlockSpec((D,), lambda i: (0,)),
          pl.BlockSpec((D, n_classes), lambda i: (0, 0)),
          pl.BlockSpec((n_classes,), lambda i: (0,)),
      ],
      out_specs=pl.BlockSpec((1, n_classes), lambda i: (0, 0)),
      out_shape=jax.ShapeDtypeStruct((1, n_classes), _f32),
      scratch_shapes=[pltpu.VMEM((1, D), _f32)],
  )(p0, p1, nd, w2, b2, wl, bl)


def kernel(x, edge_index, W1, b1, W2, b2, Wl, bl):
  src2 = edge_index[0].reshape(E // CW, CW)
  dst2 = edge_index[1].reshape(E // CW, CW)

  ds0, ds1, dd0, dd1 = _sc_degrees(src2, dst2)
  h0, ns, nd = _tc_prep(ds0, ds1, dd0, dd1, x, W1)
  p10, p11 = _sc_edge_pass(h0, src2, dst2)
  h1s = _tc_mid(p10, p11, nd, ns, b1)
  p20, p21 = _sc_edge_pass(h1s, src2, dst2)
  return _tc_final(p20, p21, nd, W2, b2, Wl, bl)
